# baseline (speedup 1.0000x reference)
import functools

import jax
import jax.numpy as jnp
from jax import lax
from jax.experimental import pallas as pl
from jax.experimental.pallas import tpu as pltpu
from jax.experimental.pallas import tpu_sc as plsc

N = 10000
D = 128
NC, NS = 2, 16
NW = NC * NS
CHUNK = 128
N_PAD = 10240
ROWS_PER_SUB = N_PAD // NS


def _sc_aggregate(table, src1d, dst1d, zeros2d, zeros1d, ones1d,
                  with_counts):
  n_chunks = src1d.shape[0] // (NW * CHUNK)
  mesh = plsc.VectorSubcoreMesh(core_axis_name="c", subcore_axis_name="s")

  out_type = [jax.ShapeDtypeStruct((NC * N_PAD, D), jnp.float32)]
  if with_counts:
    out_type.append(jax.ShapeDtypeStruct((NC * N_PAD,), jnp.float32))

  scratch = [
      pltpu.VMEM_SHARED((N_PAD, D), jnp.float32),
      pltpu.VMEM_SHARED((N_PAD,), jnp.float32),
      pltpu.VMEM((CHUNK,), jnp.int32),
      pltpu.VMEM((CHUNK,), jnp.int32),
      pltpu.VMEM((CHUNK,), jnp.int32),
      pltpu.VMEM((CHUNK,), jnp.int32),
      pltpu.VMEM((CHUNK, D), jnp.float32),
      pltpu.VMEM((CHUNK, D), jnp.float32),
      pltpu.VMEM((CHUNK,), jnp.float32),
      pltpu.SemaphoreType.DMA,
      pltpu.SemaphoreType.DMA,
      pltpu.SemaphoreType.DMA,
      pltpu.SemaphoreType.DMA,
  ]

  def body(table_h, src_h, dst_h, z2_h, z1_h, ones_h, *rest):
    if with_counts:
      sums_out, cnt_out = rest[0], rest[1]
      rest = rest[2:]
    else:
      sums_out, cnt_out = rest[0], None
      rest = rest[1:]
    (acc, cnt_acc, src0, src1, dst0, dst1, rows0, rows1, ones_v,
     gsem0, gsem1, isem0, isem1) = rest

    cid = lax.axis_index("c")
    sid = lax.axis_index("s")
    wid = cid * NS + sid
    stripe = sid * ROWS_PER_SUB
    base = wid * n_chunks * CHUNK

    pltpu.sync_copy(z2_h, acc.at[pl.ds(stripe, ROWS_PER_SUB)])
    pltpu.sync_copy(z1_h, cnt_acc.at[pl.ds(stripe, ROWS_PER_SUB)])
    pltpu.sync_copy(ones_h, ones_v)
    plsc.subcore_barrier()

    srcs, dsts, rows = (src0, src1), (dst0, dst1), (rows0, rows1)
    gsems, isems = (gsem0, gsem1), (isem0, isem1)

    def start_idx(c, s):
      off = base + c * CHUNK
      pltpu.async_copy(src_h.at[pl.ds(off, CHUNK)], srcs[s], isems[s])
      pltpu.async_copy(dst_h.at[pl.ds(off, CHUNK)], dsts[s], isems[s])

    def wait_idx(c, s):
      off = base + c * CHUNK
      pltpu.make_async_copy(src_h.at[pl.ds(off, CHUNK)], srcs[s],
                            isems[s]).wait()
      pltpu.make_async_copy(dst_h.at[pl.ds(off, CHUNK)], dsts[s],
                            isems[s]).wait()

    def phase(cb, b, has_next2):
      wait_idx(cb, b)
      pltpu.async_copy(table_h.at[srcs[b]], rows[b], gsems[b]).wait()
      pltpu.sync_copy(rows[b], acc.at[dsts[b]], add=True)
      if with_counts:
        pltpu.sync_copy(ones_v, cnt_acc.at[dsts[b]], add=True)
      if has_next2:
        start_idx(cb + 2, b)

    start_idx(0, 0)
    start_idx(1, 1)

    @pl.loop(0, n_chunks - 2, step=2)
    def _chunk(c):
      phase(c, 0, True)
      phase(c + 1, 1, True)

    phase(n_chunks - 2, 0, False)
    phase(n_chunks - 1, 1, False)

    plsc.subcore_barrier()
    out_off = cid * N_PAD + stripe
    pltpu.sync_copy(acc.at[pl.ds(stripe, ROWS_PER_SUB)],
                    sums_out.at[pl.ds(out_off, ROWS_PER_SUB)])
    if with_counts:
      pltpu.sync_copy(cnt_acc.at[pl.ds(stripe, ROWS_PER_SUB)],
                      cnt_out.at[pl.ds(out_off, ROWS_PER_SUB)])

  fn = pl.kernel(body, out_type=tuple(out_type), mesh=mesh,
                 scratch_types=scratch)
  return fn(table, src1d, dst1d, zeros2d, zeros1d, ones1d)


def _dense_body(s0_ref, s1_ref, c_ref, x_ref, wl_ref, bl_ref, wr_ref, o_ref):
  c = c_ref[:, 0] + c_ref[:, 1]
  inv = 1.0 / jnp.maximum(c, 1.0)
  mean = (s0_ref[...] + s1_ref[...]) * inv[:, None]
  out = (jnp.dot(mean, wl_ref[...], preferred_element_type=jnp.float32)
         + jnp.dot(x_ref[...], wr_ref[...], preferred_element_type=jnp.float32)
         + bl_ref[...])
  nrm = jnp.sqrt(jnp.sum(out * out, axis=-1, keepdims=True))
  out = out / jnp.maximum(nrm, 1e-12)
  o_ref[...] = jnp.where(out > 0, out, jnp.exp(out) - 1.0)


def _tc_dense(s0, s1, cpair, x, wl, bl, wr):
  rows = 1000
  grid = (N // rows,)
  return pl.pallas_call(
      _dense_body,
      grid=grid,
      in_specs=[
          pl.BlockSpec((rows, D), lambda i: (i, 0)),
          pl.BlockSpec((rows, D), lambda i: (i, 0)),
          pl.BlockSpec((rows, 2), lambda i: (i, 0)),
          pl.BlockSpec((rows, D), lambda i: (i, 0)),
          pl.BlockSpec((D, D), lambda i: (0, 0)),
          pl.BlockSpec((1, D), lambda i: (0, 0)),
          pl.BlockSpec((D, D), lambda i: (0, 0)),
      ],
      out_specs=pl.BlockSpec((rows, D), lambda i: (i, 0)),
      out_shape=jax.ShapeDtypeStruct((N, D), jnp.float32),
  )(s0, s1, cpair, x, wl, bl, wr)


def kernel(x, edge_index, Wl1, bl1, Wr1, Wl2, bl2, Wr2):
  src = edge_index[0]
  dst = edge_index[1]
  e = src.shape[0]
  n_chunks = -(-e // (NW * CHUNK))
  n_chunks += n_chunks % 2
  e_pad = NW * CHUNK * n_chunks
  pad = e_pad - e
  src_p = jnp.concatenate([src, jnp.zeros((pad,), jnp.int32)])
  dst_p = jnp.concatenate([dst, jnp.full((pad,), N, jnp.int32)])
  z2 = jnp.zeros((ROWS_PER_SUB, D), jnp.float32)
  z1 = jnp.zeros((ROWS_PER_SUB,), jnp.float32)
  ones = jnp.ones((CHUNK,), jnp.float32)
  bl1r = bl1.reshape(1, D)
  bl2r = bl2.reshape(1, D)

  sums1, cnt = _sc_aggregate(x, src_p, dst_p, z2, z1, ones, True)
  cpair = jnp.stack([cnt[:N], cnt[N_PAD:N_PAD + N]], axis=1)
  h1 = _tc_dense(sums1[:N], sums1[N_PAD:N_PAD + N], cpair, x, Wl1, bl1r, Wr1)

  (sums2,) = _sc_aggregate(h1, src_p, dst_p, z2, z1, ones, False)
  h2 = _tc_dense(sums2[:N], sums2[N_PAD:N_PAD + N], cpair, h1, Wl2, bl2r, Wr2)
  return h2


# speedup vs baseline: 1.4820x; 1.4820x over previous
import functools

import jax
import jax.numpy as jnp
from jax import lax
from jax.experimental import pallas as pl
from jax.experimental.pallas import tpu as pltpu
from jax.experimental.pallas import tpu_sc as plsc

N = 10000
D = 128
NC, NS = 2, 16
NW = NC * NS
CHUNK = 128
N_PAD = 10240
ROWS_PER_SUB = N_PAD // NS


def _sc_aggregate(table, edges_il, zeros2d, zeros1d, ones1d, with_counts):
  n_chunks = edges_il.shape[0] // (2 * NW)
  mesh = plsc.VectorSubcoreMesh(core_axis_name="c", subcore_axis_name="s")

  out_type = [jax.ShapeDtypeStruct((NC * N_PAD, D), jnp.float32)]
  if with_counts:
    out_type.append(jax.ShapeDtypeStruct((NC * N_PAD,), jnp.float32))

  scratch = [
      pltpu.VMEM_SHARED((N_PAD, D), jnp.float32),
      pltpu.VMEM_SHARED((N_PAD,), jnp.float32),
      pltpu.VMEM((2, CHUNK), jnp.int32),
      pltpu.VMEM((CHUNK, D), jnp.float32),
      pltpu.VMEM((CHUNK,), jnp.float32),
      pltpu.SemaphoreType.DMA,
  ]

  def body(table_h, edge_h, z2_h, z1_h, ones_h, *rest):
    if with_counts:
      sums_out, cnt_out = rest[0], rest[1]
      rest = rest[2:]
    else:
      sums_out, cnt_out = rest[0], None
      rest = rest[1:]
    acc, cnt_acc, idx_v, rows_v, ones_v, sem = rest

    cid = lax.axis_index("c")
    sid = lax.axis_index("s")
    wid = cid * NS + sid
    stripe = sid * ROWS_PER_SUB
    base = wid * n_chunks * 2

    pltpu.sync_copy(z2_h, acc.at[pl.ds(stripe, ROWS_PER_SUB)])
    pltpu.sync_copy(z1_h, cnt_acc.at[pl.ds(stripe, ROWS_PER_SUB)])
    pltpu.sync_copy(ones_h, ones_v)
    plsc.subcore_barrier()

    @pl.loop(0, n_chunks)
    def _chunk(c):
      pltpu.sync_copy(edge_h.at[pl.ds(base + c * 2, 2)], idx_v)
      pltpu.async_copy(table_h.at[idx_v.at[0]], rows_v, sem).wait()
      pltpu.sync_copy(rows_v, acc.at[idx_v.at[1]], add=True)
      if with_counts:
        pltpu.sync_copy(ones_v, cnt_acc.at[idx_v.at[1]], add=True)

    plsc.subcore_barrier()
    out_off = cid * N_PAD + stripe
    pltpu.sync_copy(acc.at[pl.ds(stripe, ROWS_PER_SUB)],
                    sums_out.at[pl.ds(out_off, ROWS_PER_SUB)])
    if with_counts:
      pltpu.sync_copy(cnt_acc.at[pl.ds(stripe, ROWS_PER_SUB)],
                      cnt_out.at[pl.ds(out_off, ROWS_PER_SUB)])

  fn = pl.kernel(body, out_type=tuple(out_type), mesh=mesh,
                 scratch_types=scratch)
  return fn(table, edges_il, zeros2d, zeros1d, ones1d)


def _dense_body(s0_ref, s1_ref, c_ref, x_ref, wl_ref, bl_ref, wr_ref, o_ref):
  c = c_ref[:, 0] + c_ref[:, 1]
  inv = 1.0 / jnp.maximum(c, 1.0)
  mean = (s0_ref[...] + s1_ref[...]) * inv[:, None]
  out = (jnp.dot(mean, wl_ref[...], preferred_element_type=jnp.float32)
         + jnp.dot(x_ref[...], wr_ref[...], preferred_element_type=jnp.float32)
         + bl_ref[...])
  nrm = jnp.sqrt(jnp.sum(out * out, axis=-1, keepdims=True))
  out = out / jnp.maximum(nrm, 1e-12)
  o_ref[...] = jnp.where(out > 0, out, jnp.exp(out) - 1.0)


def _tc_dense(s0, s1, cpair, x, wl, bl, wr):
  rows = 1000
  grid = (N // rows,)
  return pl.pallas_call(
      _dense_body,
      grid=grid,
      in_specs=[
          pl.BlockSpec((rows, D), lambda i: (i, 0)),
          pl.BlockSpec((rows, D), lambda i: (i, 0)),
          pl.BlockSpec((rows, 2), lambda i: (i, 0)),
          pl.BlockSpec((rows, D), lambda i: (i, 0)),
          pl.BlockSpec((D, D), lambda i: (0, 0)),
          pl.BlockSpec((1, D), lambda i: (0, 0)),
          pl.BlockSpec((D, D), lambda i: (0, 0)),
      ],
      out_specs=pl.BlockSpec((rows, D), lambda i: (i, 0)),
      out_shape=jax.ShapeDtypeStruct((N, D), jnp.float32),
  )(s0, s1, cpair, x, wl, bl, wr)


def kernel(x, edge_index, Wl1, bl1, Wr1, Wl2, bl2, Wr2):
  src = edge_index[0]
  dst = edge_index[1]
  e = src.shape[0]
  n_chunks = -(-e // (NW * CHUNK))
  e_pad = NW * CHUNK * n_chunks
  pad = e_pad - e
  src_p = jnp.concatenate([src, jnp.zeros((pad,), jnp.int32)])
  dst_p = jnp.concatenate([dst, jnp.full((pad,), N, jnp.int32)])
  edges_il = jnp.stack(
      [src_p.reshape(-1, CHUNK), dst_p.reshape(-1, CHUNK)], axis=1
  ).reshape(-1, CHUNK)
  z2 = jnp.zeros((ROWS_PER_SUB, D), jnp.float32)
  z1 = jnp.zeros((ROWS_PER_SUB,), jnp.float32)
  ones = jnp.ones((CHUNK,), jnp.float32)
  bl1r = bl1.reshape(1, D)
  bl2r = bl2.reshape(1, D)

  sums1, cnt = _sc_aggregate(x, edges_il, z2, z1, ones, True)
  cpair = jnp.stack([cnt[:N], cnt[N_PAD:N_PAD + N]], axis=1)
  h1 = _tc_dense(sums1[:N], sums1[N_PAD:N_PAD + N], cpair, x, Wl1, bl1r, Wr1)

  (sums2,) = _sc_aggregate(h1, edges_il, z2, z1, ones, False)
  h2 = _tc_dense(sums2[:N], sums2[N_PAD:N_PAD + N], cpair, h1, Wl2, bl2r, Wr2)
  return h2


# speedup vs baseline: 1.9769x; 1.3339x over previous
import functools

import jax
import jax.numpy as jnp
from jax import lax
from jax.experimental import pallas as pl
from jax.experimental.pallas import tpu as pltpu
from jax.experimental.pallas import tpu_sc as plsc

N = 10000
D = 128
NC, NS = 2, 16
NW = NC * NS
CHUNK = 128
N_PAD = 10240
ROWS_PER_SUB = N_PAD // NS


def _sc_aggregate(table, edges_il, zeros2d, zeros1d, ones1d, with_counts,
                  k_sc0, k_sc1):
  mesh = plsc.VectorSubcoreMesh(core_axis_name="c", subcore_axis_name="s")

  out_type = [jax.ShapeDtypeStruct((NC * N_PAD, D), jnp.float32)]
  if with_counts:
    out_type.append(jax.ShapeDtypeStruct((NC * N_PAD,), jnp.float32))

  scratch = [
      pltpu.VMEM_SHARED((N_PAD, D), jnp.float32),
      pltpu.VMEM_SHARED((N_PAD,), jnp.float32),
      pltpu.VMEM((2, CHUNK), jnp.int32),
      pltpu.VMEM((CHUNK, D), jnp.float32),
      pltpu.VMEM((CHUNK,), jnp.float32),
      pltpu.SemaphoreType.DMA,
  ]

  def body(table_h, edge_h, z2_h, z1_h, ones_h, *rest):
    if with_counts:
      sums_out, cnt_out = rest[0], rest[1]
      rest = rest[2:]
    else:
      sums_out, cnt_out = rest[0], None
      rest = rest[1:]
    acc, cnt_acc, idx_v, rows_v, ones_v, sem = rest

    cid = lax.axis_index("c")
    sid = lax.axis_index("s")
    stripe = sid * ROWS_PER_SUB
    k_self = jnp.where(cid == 0, k_sc0, k_sc1)
    base = jnp.where(cid == 0, sid * k_sc0, NS * k_sc0 + sid * k_sc1) * 2

    pltpu.sync_copy(z2_h, acc.at[pl.ds(stripe, ROWS_PER_SUB)])
    pltpu.sync_copy(z1_h, cnt_acc.at[pl.ds(stripe, ROWS_PER_SUB)])
    pltpu.sync_copy(ones_h, ones_v)
    plsc.subcore_barrier()

    @pl.loop(0, k_self)
    def _chunk(c):
      pltpu.sync_copy(edge_h.at[pl.ds(base + c * 2, 2)], idx_v)
      pltpu.async_copy(table_h.at[idx_v.at[0]], rows_v, sem).wait()
      pltpu.sync_copy(rows_v, acc.at[idx_v.at[1]], add=True)
      if with_counts:
        pltpu.sync_copy(ones_v, cnt_acc.at[idx_v.at[1]], add=True)

    plsc.subcore_barrier()
    out_off = cid * N_PAD + stripe
    pltpu.sync_copy(acc.at[pl.ds(stripe, ROWS_PER_SUB)],
                    sums_out.at[pl.ds(out_off, ROWS_PER_SUB)])
    if with_counts:
      pltpu.sync_copy(cnt_acc.at[pl.ds(stripe, ROWS_PER_SUB)],
                      cnt_out.at[pl.ds(out_off, ROWS_PER_SUB)])

  fn = pl.kernel(body, out_type=tuple(out_type), mesh=mesh,
                 scratch_types=scratch)
  return fn(table, edges_il, zeros2d, zeros1d, ones1d)


SC0_SHARE = 0.61


def _dense_body(s0_ref, s1_ref, c_ref, x_ref, wl_ref, bl_ref, wr_ref, o_ref):
  c = c_ref[:, 0] + c_ref[:, 1]
  inv = 1.0 / jnp.maximum(c, 1.0)
  mean = (s0_ref[...] + s1_ref[...]) * inv[:, None]
  out = (jnp.dot(mean, wl_ref[...], preferred_element_type=jnp.float32)
         + jnp.dot(x_ref[...], wr_ref[...], preferred_element_type=jnp.float32)
         + bl_ref[...])
  nrm = jnp.sqrt(jnp.sum(out * out, axis=-1, keepdims=True))
  out = out / jnp.maximum(nrm, 1e-12)
  o_ref[...] = jnp.where(out > 0, out, jnp.exp(out) - 1.0)


def _tc_dense(s0, s1, cpair, x, wl, bl, wr):
  rows = 1000
  grid = (N // rows,)
  return pl.pallas_call(
      _dense_body,
      grid=grid,
      in_specs=[
          pl.BlockSpec((rows, D), lambda i: (i, 0)),
          pl.BlockSpec((rows, D), lambda i: (i, 0)),
          pl.BlockSpec((rows, 2), lambda i: (i, 0)),
          pl.BlockSpec((rows, D), lambda i: (i, 0)),
          pl.BlockSpec((D, D), lambda i: (0, 0)),
          pl.BlockSpec((1, D), lambda i: (0, 0)),
          pl.BlockSpec((D, D), lambda i: (0, 0)),
      ],
      out_specs=pl.BlockSpec((rows, D), lambda i: (i, 0)),
      out_shape=jax.ShapeDtypeStruct((N, D), jnp.float32),
  )(s0, s1, cpair, x, wl, bl, wr)


def kernel(x, edge_index, Wl1, bl1, Wr1, Wl2, bl2, Wr2):
  src = edge_index[0]
  dst = edge_index[1]
  e = src.shape[0]
  k_pair = -(-e // (NS * CHUNK))
  k_sc0 = max(1, min(k_pair - 1, round(k_pair * SC0_SHARE)))
  k_sc1 = k_pair - k_sc0
  e_pad = NS * k_pair * CHUNK
  pad = e_pad - e
  src_p = jnp.concatenate([src, jnp.zeros((pad,), jnp.int32)])
  dst_p = jnp.concatenate([dst, jnp.full((pad,), N, jnp.int32)])
  edges_il = jnp.stack(
      [src_p.reshape(-1, CHUNK), dst_p.reshape(-1, CHUNK)], axis=1
  ).reshape(-1, CHUNK)
  z2 = jnp.zeros((ROWS_PER_SUB, D), jnp.float32)
  z1 = jnp.zeros((ROWS_PER_SUB,), jnp.float32)
  ones = jnp.ones((CHUNK,), jnp.float32)
  bl1r = bl1.reshape(1, D)
  bl2r = bl2.reshape(1, D)

  sums1, cnt = _sc_aggregate(x, edges_il, z2, z1, ones, True, k_sc0, k_sc1)
  cpair = jnp.stack([cnt[:N], cnt[N_PAD:N_PAD + N]], axis=1)
  h1 = _tc_dense(sums1[:N], sums1[N_PAD:N_PAD + N], cpair, x, Wl1, bl1r, Wr1)

  (sums2,) = _sc_aggregate(h1, edges_il, z2, z1, ones, False, k_sc0, k_sc1)
  h2 = _tc_dense(sums2[:N], sums2[N_PAD:N_PAD + N], cpair, h1, Wl2, bl2r, Wr2)
  return h2


# speedup vs baseline: 2.0771x; 1.0507x over previous
import functools

import jax
import jax.numpy as jnp
from jax import lax
from jax.experimental import pallas as pl
from jax.experimental.pallas import tpu as pltpu
from jax.experimental.pallas import tpu_sc as plsc

N = 10000
D = 128
NC, NS = 2, 16
NW = NC * NS
CHUNK = 128
N_PAD = 10240
ROWS_PER_SUB = N_PAD // NS


def _sc_aggregate(table, edges_il, zeros2d, zeros1d, ones1d, with_counts,
                  k_sc0, k_sc1):
  mesh = plsc.VectorSubcoreMesh(core_axis_name="c", subcore_axis_name="s")

  out_type = [jax.ShapeDtypeStruct((NC * N_PAD, D), jnp.float32)]
  if with_counts:
    out_type.append(jax.ShapeDtypeStruct((NC * N_PAD,), jnp.float32))

  scratch = [
      pltpu.VMEM_SHARED((N_PAD, D), jnp.float32),
      pltpu.VMEM_SHARED((N_PAD,), jnp.float32),
      pltpu.VMEM((2, CHUNK), jnp.int32),
      pltpu.VMEM((CHUNK, D), jnp.float32),
      pltpu.VMEM((CHUNK,), jnp.float32),
      pltpu.SemaphoreType.DMA,
  ]

  def body(table_h, edge_h, z2_h, z1_h, ones_h, *rest):
    if with_counts:
      sums_out, cnt_out = rest[0], rest[1]
      rest = rest[2:]
    else:
      sums_out, cnt_out = rest[0], None
      rest = rest[1:]
    acc, cnt_acc, idx_v, rows_v, ones_v, sem = rest

    cid = lax.axis_index("c")
    sid = lax.axis_index("s")
    stripe = sid * ROWS_PER_SUB
    k_self = jnp.where(cid == 0, k_sc0, k_sc1)
    base = jnp.where(cid == 0, sid * k_sc0, NS * k_sc0 + sid * k_sc1) * 2

    pltpu.sync_copy(z2_h, acc.at[pl.ds(stripe, ROWS_PER_SUB)])
    pltpu.sync_copy(z1_h, cnt_acc.at[pl.ds(stripe, ROWS_PER_SUB)])
    pltpu.sync_copy(ones_h, ones_v)
    plsc.subcore_barrier()

    @pl.loop(0, k_self)
    def _chunk(c):
      pltpu.sync_copy(edge_h.at[pl.ds(base + c * 2, 2)], idx_v)
      pltpu.async_copy(table_h.at[idx_v.at[0]], rows_v, sem).wait()
      pltpu.sync_copy(rows_v, acc.at[idx_v.at[1]], add=True)
      if with_counts:
        pltpu.sync_copy(ones_v, cnt_acc.at[idx_v.at[1]], add=True)

    plsc.subcore_barrier()
    out_off = cid * N_PAD + stripe
    pltpu.sync_copy(acc.at[pl.ds(stripe, ROWS_PER_SUB)],
                    sums_out.at[pl.ds(out_off, ROWS_PER_SUB)])
    if with_counts:
      pltpu.sync_copy(cnt_acc.at[pl.ds(stripe, ROWS_PER_SUB)],
                      cnt_out.at[pl.ds(out_off, ROWS_PER_SUB)])

  fn = pl.kernel(body, out_type=tuple(out_type), mesh=mesh,
                 scratch_types=scratch)
  return fn(table, edges_il, zeros2d, zeros1d, ones1d)


SC0_SHARE = 0.575


def _dense_body(s0_ref, s1_ref, c_ref, x_ref, wl_ref, bl_ref, wr_ref, o_ref):
  c = c_ref[:, 0] + c_ref[:, 1]
  inv = 1.0 / jnp.maximum(c, 1.0)
  mean = (s0_ref[...] + s1_ref[...]) * inv[:, None]
  out = (jnp.dot(mean, wl_ref[...], preferred_element_type=jnp.float32)
         + jnp.dot(x_ref[...], wr_ref[...], preferred_element_type=jnp.float32)
         + bl_ref[...])
  nrm = jnp.sqrt(jnp.sum(out * out, axis=-1, keepdims=True))
  out = out / jnp.maximum(nrm, 1e-12)
  o_ref[...] = jnp.where(out > 0, out, jnp.exp(out) - 1.0)


def _tc_dense(s0, s1, cpair, x, wl, bl, wr):
  rows = 1000
  grid = (N // rows,)
  return pl.pallas_call(
      _dense_body,
      grid=grid,
      in_specs=[
          pl.BlockSpec((rows, D), lambda i: (i, 0)),
          pl.BlockSpec((rows, D), lambda i: (i, 0)),
          pl.BlockSpec((rows, 2), lambda i: (i, 0)),
          pl.BlockSpec((rows, D), lambda i: (i, 0)),
          pl.BlockSpec((D, D), lambda i: (0, 0)),
          pl.BlockSpec((1, D), lambda i: (0, 0)),
          pl.BlockSpec((D, D), lambda i: (0, 0)),
      ],
      out_specs=pl.BlockSpec((rows, D), lambda i: (i, 0)),
      out_shape=jax.ShapeDtypeStruct((N, D), jnp.float32),
  )(s0, s1, cpair, x, wl, bl, wr)


def kernel(x, edge_index, Wl1, bl1, Wr1, Wl2, bl2, Wr2):
  src = edge_index[0]
  dst = edge_index[1]
  e = src.shape[0]
  k_pair = -(-e // (NS * CHUNK))
  k_sc0 = max(1, min(k_pair - 1, round(k_pair * SC0_SHARE)))
  k_sc1 = k_pair - k_sc0
  e_pad = NS * k_pair * CHUNK
  pad = e_pad - e
  src_p = jnp.concatenate([src, jnp.zeros((pad,), jnp.int32)])
  dst_p = jnp.concatenate([dst, jnp.full((pad,), N, jnp.int32)])
  edges_il = jnp.stack(
      [src_p.reshape(-1, CHUNK), dst_p.reshape(-1, CHUNK)], axis=1
  ).reshape(-1, CHUNK)
  z2 = jnp.zeros((ROWS_PER_SUB, D), jnp.float32)
  z1 = jnp.zeros((ROWS_PER_SUB,), jnp.float32)
  ones = jnp.ones((CHUNK,), jnp.float32)
  bl1r = bl1.reshape(1, D)
  bl2r = bl2.reshape(1, D)

  sums1, cnt = _sc_aggregate(x, edges_il, z2, z1, ones, True, k_sc0, k_sc1)
  cpair = jnp.stack([cnt[:N], cnt[N_PAD:N_PAD + N]], axis=1)
  h1 = _tc_dense(sums1[:N], sums1[N_PAD:N_PAD + N], cpair, x, Wl1, bl1r, Wr1)

  (sums2,) = _sc_aggregate(h1, edges_il, z2, z1, ones, False, k_sc0, k_sc1)
  h2 = _tc_dense(sums2[:N], sums2[N_PAD:N_PAD + N], cpair, h1, Wl2, bl2r, Wr2)
  return h2
